# Initial kernel scaffold; baseline (speedup 1.0000x reference)
import jax
import jax.numpy as jnp
from jax.experimental import pallas as pl


def kernel(h, c, obs, W, b):
    raise NotImplementedError("write your pallas kernel here")


# speedup vs baseline: 105.6857x; 105.6857x over previous
import functools

import jax
import jax.numpy as jnp
from jax.experimental import pallas as pl
from jax.experimental.pallas import tpu as pltpu

_N = 4096
_NG = 6
_HID = 128
_TI = 128
_SJ = 512


def _occ_kernel(xj_ref, yj_ref, oiT_ref, w_ref, b_ref, out_ref):
    t = pl.program_id(0)
    xi = oiT_ref[0:1, :]
    yi = oiT_ref[1:2, :]
    i_ids = t * _TI + jax.lax.broadcasted_iota(jnp.int32, (1, _TI), 1)
    j_iota = jax.lax.broadcasted_iota(jnp.int32, (_SJ, 1), 0)

    def body(c, accs):
        base = c * _SJ
        sx = xj_ref[pl.ds(base, _SJ), :]
        sy = yj_ref[pl.ds(base, _SJ), :]
        relx = (sx - xi) * 2.0 + 3.0
        rely = (sy - yi) * 2.0 + 3.0
        xbf = jnp.floor(relx)
        ybf = jnp.floor(rely)
        vy = (ybf >= 0.0) & (ybf <= 5.0)
        ne = (base + j_iota) != i_ids
        vm = vy & ne
        yi_int = jnp.where(vy, ybf, 0.0).astype(jnp.int32)
        val = jnp.where(vm, jnp.int32(1) << yi_int, 0)
        return tuple(
            accs[bx] | jnp.where(xbf == float(bx), val, 0)
            for bx in range(_NG)
        )

    zero = jnp.zeros((_SJ, _TI), jnp.int32)
    accs = jax.lax.fori_loop(0, _N // _SJ, body, (zero,) * _NG)

    occ_rows = []
    for bx in range(_NG):
        a = accs[bx]
        s = _SJ
        while s > 8:
            h = s // 2
            a = a[:h] | a[h:s]
            s = h
        for by in range(_NG):
            bit = (a >> by) & 1
            occ_rows.append(
                jnp.max(bit, axis=0, keepdims=True).astype(jnp.float32))
    occT = jnp.concatenate(occ_rows, axis=0)

    out = jax.lax.dot_general(
        occT, w_ref[...],
        dimension_numbers=(((0,), (1,)), ((), ())),
        preferred_element_type=jnp.float32)
    out_ref[...] = out + b_ref[...]


@functools.partial(jax.jit, static_argnames=())
def kernel(h, c, obs, W, b):
    del h, c
    obs = obs.astype(jnp.float32)
    xj = obs[:, 0:1]
    yj = obs[:, 1:2]
    oiT = jnp.concatenate(
        [obs.T, jnp.zeros((6, _N), jnp.float32)], axis=0)
    b2 = b.reshape(1, _HID).astype(jnp.float32)

    grid = (_N // _TI,)
    out = pl.pallas_call(
        _occ_kernel,
        grid=grid,
        in_specs=[
            pl.BlockSpec((_N, 1), lambda t: (0, 0)),
            pl.BlockSpec((_N, 1), lambda t: (0, 0)),
            pl.BlockSpec((8, _TI), lambda t: (0, t)),
            pl.BlockSpec((_HID, _NG * _NG), lambda t: (0, 0)),
            pl.BlockSpec((1, _HID), lambda t: (0, 0)),
        ],
        out_specs=pl.BlockSpec((_TI, _HID), lambda t: (t, 0)),
        out_shape=jax.ShapeDtypeStruct((_N, _HID), jnp.float32),
        compiler_params=pltpu.CompilerParams(
            dimension_semantics=("parallel",)),
    )(xj, yj, oiT, W.astype(jnp.float32), b2)
    return out
